# baseline (speedup 1.0000x reference)
import jax
import jax.numpy as jnp
from jax.experimental import pallas as pl
from jax.experimental.pallas import tpu as pltpu


def _blk(x_ref, o_ref):
    ids = jax.lax.broadcasted_iota(jnp.int32, o_ref.shape, 1)
    o_ref[...] = (ids == x_ref[...]).astype(o_ref.dtype)


def kernel(x):
    out_dtype = jnp.zeros((), jnp.int64).dtype
    x2 = x.reshape(51200, 1).astype(jnp.int32)[:50000]
    out = pl.pallas_call(
        _blk,
        grid=(50,),
        in_specs=[pl.BlockSpec((1000, 1), lambda i: (i, 0))],
        out_specs=pl.BlockSpec((1000, 1024), lambda i: (i, 0)),
        out_shape=jax.ShapeDtypeStruct((50000, 1024), out_dtype),
    )(x2)
    return out.reshape(1024, 50, 1000)


# speedup vs baseline: 8.5509x; 8.5509x over previous
import jax
import jax.numpy as jnp
from jax.experimental import pallas as pl
from jax.experimental.pallas import tpu as pltpu


def _blk(x_ref, o_ref):
    ids = jax.lax.broadcasted_iota(jnp.int32, o_ref.shape, 1)
    o_ref[...] = (ids == x_ref[...]).astype(o_ref.dtype)


def kernel(x):
    out_dtype = jnp.zeros((), jnp.int64).dtype
    x2 = x.reshape(51200, 1).astype(jnp.int32)[:50000]
    out = pl.pallas_call(
        _blk,
        grid=(50,),
        in_specs=[pl.BlockSpec((1000, 1), lambda i: (i, 0))],
        out_specs=pl.BlockSpec((1000, 1024), lambda i: (i, 0)),
        out_shape=jax.ShapeDtypeStruct((50000, 1024), out_dtype),
    )(x2)
    return out


# speedup vs baseline: 13.1366x; 1.5363x over previous
import jax
import jax.numpy as jnp
from jax.experimental import pallas as pl
from jax.experimental.pallas import tpu as pltpu

B_ = 1024
S_ = 50
NUM_CLASSES_ = 1000


def _onehot_block(x_ref, o_ref):
    ids = jax.lax.broadcasted_iota(jnp.int32, o_ref.shape, 1)
    o_ref[...] = (ids == x_ref[...]).astype(o_ref.dtype)


def kernel(x):
    out_dtype = jnp.zeros((), jnp.int64).dtype
    xt = jnp.transpose(x).astype(jnp.int32).reshape(S_, 1, B_)
    out_t = pl.pallas_call(
        _onehot_block,
        grid=(S_,),
        in_specs=[pl.BlockSpec((1, 1, B_), lambda i: (i, 0, 0))],
        out_specs=pl.BlockSpec((1, NUM_CLASSES_, B_), lambda i: (i, 0, 0)),
        out_shape=jax.ShapeDtypeStruct((S_, NUM_CLASSES_, B_), out_dtype),
    )(xt)
    return jnp.transpose(out_t, (2, 0, 1))
